# baseline (speedup 1.0000x reference)
import jax
import jax.numpy as jnp
from jax.experimental import pallas as pl
from jax.experimental.pallas import tpu as pltpu

_BITS = 8
_EBIT = 8


def _quant_block(x_ref, o_ref):
    x = x_ref[...]
    d = jnp.where(x >= 0, jnp.clip(x, 1e-10, None), jnp.clip(x, None, -1e-10))
    m = jnp.max(jnp.abs(d), axis=1, keepdims=True)
    e = jnp.floor(jnp.log2(m))
    e = jnp.clip(e, -(2.0 ** (_EBIT - 1)), 2.0 ** (_EBIT - 1) - 1)
    i = jnp.round(d * jnp.exp2((_BITS - 2) - e))
    i = jnp.clip(i, -(2.0 ** (_BITS - 1)), 2.0 ** (_BITS - 1) - 1)
    o_ref[...] = i * jnp.exp2(e - (_BITS - 2))


def kernel(x):
    B, N = x.shape
    R = 128
    return pl.pallas_call(
        _quant_block,
        grid=(B // R,),
        in_specs=[pl.BlockSpec((R, N), lambda i: (i, 0))],
        out_specs=pl.BlockSpec((R, N), lambda i: (i, 0)),
        out_shape=jax.ShapeDtypeStruct((B, N), x.dtype),
        compiler_params=pltpu.CompilerParams(
            dimension_semantics=("parallel",),
        ),
    )(x)


# speedup vs baseline: 1.0478x; 1.0478x over previous
import jax
import jax.numpy as jnp
from jax.experimental import pallas as pl
from jax.experimental.pallas import tpu as pltpu

_BITS = 8
_EBIT = 8


def _quant_block(x_ref, o_ref):
    x = x_ref[...]
    d = jnp.where(x >= 0, jnp.clip(x, 1e-10, None), jnp.clip(x, None, -1e-10))
    m = jnp.max(jnp.abs(d), axis=1, keepdims=True)
    e = jnp.floor(jnp.log2(m))
    e = jnp.clip(e, -(2.0 ** (_EBIT - 1)), 2.0 ** (_EBIT - 1) - 1)
    i = jnp.round(d * jnp.exp2((_BITS - 2) - e))
    i = jnp.clip(i, -(2.0 ** (_BITS - 1)), 2.0 ** (_BITS - 1) - 1)
    o_ref[...] = i * jnp.exp2(e - (_BITS - 2))


def kernel(x):
    B, N = x.shape
    R = 256
    return pl.pallas_call(
        _quant_block,
        grid=(B // R,),
        in_specs=[pl.BlockSpec((R, N), lambda i: (i, 0))],
        out_specs=pl.BlockSpec((R, N), lambda i: (i, 0)),
        out_shape=jax.ShapeDtypeStruct((B, N), x.dtype),
        compiler_params=pltpu.CompilerParams(
            dimension_semantics=("parallel",),
        ),
    )(x)
